# speedup vs baseline: 3.3690x; 3.3690x over previous
import jax
import jax.numpy as jnp
from jax.experimental import pallas as pl
import jax.experimental.pallas.tpu as pltpu

Q_SIZE = 1024
K_SIZE = 100000
D_SIZE = 64
K_TILE = 2000
N_TILES = K_SIZE // K_TILE


def _nn_kernel(x_ref, idx_ref, dist_ref, arg_ref, xsq_ref, min_ref, argm_ref):
    j = pl.program_id(0)

    @pl.when(j == 0)
    def _init():
        x = x_ref[...]
        xsq_ref[...] = jnp.sum(x * x, axis=1, keepdims=True)
        min_ref[...] = jnp.full((Q_SIZE, 1), jnp.inf, jnp.float32)
        argm_ref[...] = jnp.zeros((Q_SIZE, 1), jnp.int32)

    tile = idx_ref[...]
    s = jnp.sum(tile * tile, axis=1)[None, :]
    mm = jax.lax.dot_general(
        x_ref[...], tile,
        dimension_numbers=(((1,), (1,)), ((), ())),
        preferred_element_type=jnp.float32,
    )
    t = xsq_ref[...] + s
    d = t - 2.0 * mm
    d = jnp.maximum(d, 0.0)

    local_min = jnp.min(d, axis=1, keepdims=True)
    local_arg = jnp.argmin(d, axis=1).astype(jnp.int32)[:, None] + j * K_TILE

    better = local_min < min_ref[...]
    min_ref[...] = jnp.where(better, local_min, min_ref[...])
    argm_ref[...] = jnp.where(better, local_arg, argm_ref[...])

    @pl.when(j == N_TILES - 1)
    def _done():
        dist_ref[...] = min_ref[...]
        arg_ref[...] = argm_ref[...]


@jax.jit
def kernel(x, index):
    dist, arg = pl.pallas_call(
        _nn_kernel,
        grid=(N_TILES,),
        in_specs=[
            pl.BlockSpec((Q_SIZE, D_SIZE), lambda j: (0, 0)),
            pl.BlockSpec((K_TILE, D_SIZE), lambda j: (j, 0)),
        ],
        out_specs=[
            pl.BlockSpec((Q_SIZE, 1), lambda j: (0, 0)),
            pl.BlockSpec((Q_SIZE, 1), lambda j: (0, 0)),
        ],
        out_shape=[
            jax.ShapeDtypeStruct((Q_SIZE, 1), jnp.float32),
            jax.ShapeDtypeStruct((Q_SIZE, 1), jnp.int32),
        ],
        scratch_shapes=[
            pltpu.VMEM((Q_SIZE, 1), jnp.float32),
            pltpu.VMEM((Q_SIZE, 1), jnp.float32),
            pltpu.VMEM((Q_SIZE, 1), jnp.int32),
        ],
    )(x, index)
    return dist, arg


# speedup vs baseline: 4.2930x; 1.2742x over previous
import jax
import jax.numpy as jnp
from jax.experimental import pallas as pl
import jax.experimental.pallas.tpu as pltpu

Q_SIZE = 1024
K_SIZE = 100000
D_SIZE = 64
K_TILE = 2000
N_TILES = K_SIZE // K_TILE


def _nn_kernel(xm2_ref, xsq_ref, sq_ref, idx_ref, dist_ref, arg_ref,
               min_ref, argm_ref):
    j = pl.program_id(0)

    @pl.when(j == 0)
    def _init():
        min_ref[...] = jnp.full((Q_SIZE, 1), jnp.inf, jnp.float32)
        argm_ref[...] = jnp.zeros((Q_SIZE, 1), jnp.float32)

    tile = idx_ref[...]
    s = sq_ref[0]
    mm2 = jax.lax.dot_general(
        xm2_ref[...], tile,
        dimension_numbers=(((1,), (1,)), ((), ())),
        preferred_element_type=jnp.float32,
    )
    t = xsq_ref[...] + s
    d = t + mm2

    local_min = jnp.min(d, axis=1, keepdims=True)
    lane = jax.lax.broadcasted_iota(
        jnp.int32, (Q_SIZE, K_TILE), 1).astype(jnp.float32)
    local_arg = jnp.min(
        jnp.where(d == local_min, lane, jnp.float32(K_TILE)),
        axis=1, keepdims=True,
    ) + (j * K_TILE).astype(jnp.float32)

    better = local_min < min_ref[...]
    min_ref[...] = jnp.where(better, local_min, min_ref[...])
    argm_ref[...] = jnp.where(better, local_arg, argm_ref[...])

    @pl.when(j == N_TILES - 1)
    def _done():
        dist_ref[...] = jnp.maximum(min_ref[...], 0.0)
        arg_ref[...] = argm_ref[...].astype(jnp.int32)


@jax.jit
def kernel(x, index):
    xsq = jnp.sum(x * x, axis=1, keepdims=True)
    xm2 = x * -2.0
    idx_sq = jnp.sum(index * index, axis=1).reshape(N_TILES, 1, K_TILE)
    dist, arg = pl.pallas_call(
        _nn_kernel,
        grid=(N_TILES,),
        in_specs=[
            pl.BlockSpec((Q_SIZE, D_SIZE), lambda j: (0, 0)),
            pl.BlockSpec((Q_SIZE, 1), lambda j: (0, 0)),
            pl.BlockSpec((1, 1, K_TILE), lambda j: (j, 0, 0)),
            pl.BlockSpec((K_TILE, D_SIZE), lambda j: (j, 0)),
        ],
        out_specs=[
            pl.BlockSpec((Q_SIZE, 1), lambda j: (0, 0)),
            pl.BlockSpec((Q_SIZE, 1), lambda j: (0, 0)),
        ],
        out_shape=[
            jax.ShapeDtypeStruct((Q_SIZE, 1), jnp.float32),
            jax.ShapeDtypeStruct((Q_SIZE, 1), jnp.int32),
        ],
        scratch_shapes=[
            pltpu.VMEM((Q_SIZE, 1), jnp.float32),
            pltpu.VMEM((Q_SIZE, 1), jnp.float32),
        ],
    )(xm2, xsq, idx_sq, index)
    return dist, arg


# speedup vs baseline: 4.3886x; 1.0223x over previous
import jax
import jax.numpy as jnp
from jax.experimental import pallas as pl
import jax.experimental.pallas.tpu as pltpu

Q_SIZE = 1024
K_SIZE = 100000
D_SIZE = 64
K_TILE = 4000
N_TILES = K_SIZE // K_TILE


def _nn_kernel(xm2_ref, xsq_ref, sq_ref, idx_ref, dist_ref, arg_ref,
               min_ref, argm_ref):
    j = pl.program_id(0)

    @pl.when(j == 0)
    def _init():
        min_ref[...] = jnp.full((Q_SIZE, 1), jnp.inf, jnp.float32)
        argm_ref[...] = jnp.zeros((Q_SIZE, 1), jnp.float32)

    tile = idx_ref[...]
    s = sq_ref[0]
    mm2 = jax.lax.dot_general(
        xm2_ref[...], tile,
        dimension_numbers=(((1,), (1,)), ((), ())),
        preferred_element_type=jnp.float32,
    )
    t = xsq_ref[...] + s
    d = t + mm2

    local_min = jnp.min(d, axis=1, keepdims=True)
    lane = jax.lax.broadcasted_iota(
        jnp.int32, (Q_SIZE, K_TILE), 1).astype(jnp.float32)
    local_arg = jnp.min(
        jnp.where(d == local_min, lane, jnp.float32(K_TILE)),
        axis=1, keepdims=True,
    ) + (j * K_TILE).astype(jnp.float32)

    better = local_min < min_ref[...]
    min_ref[...] = jnp.where(better, local_min, min_ref[...])
    argm_ref[...] = jnp.where(better, local_arg, argm_ref[...])

    @pl.when(j == N_TILES - 1)
    def _done():
        dist_ref[...] = jnp.maximum(min_ref[...], 0.0)
        arg_ref[...] = argm_ref[...].astype(jnp.int32)


@jax.jit
def kernel(x, index):
    xsq = jnp.sum(x * x, axis=1, keepdims=True)
    xm2 = x * -2.0
    idx_sq = jnp.sum(index * index, axis=1).reshape(N_TILES, 1, K_TILE)
    dist, arg = pl.pallas_call(
        _nn_kernel,
        grid=(N_TILES,),
        in_specs=[
            pl.BlockSpec((Q_SIZE, D_SIZE), lambda j: (0, 0)),
            pl.BlockSpec((Q_SIZE, 1), lambda j: (0, 0)),
            pl.BlockSpec((1, 1, K_TILE), lambda j: (j, 0, 0)),
            pl.BlockSpec((K_TILE, D_SIZE), lambda j: (j, 0)),
        ],
        out_specs=[
            pl.BlockSpec((Q_SIZE, 1), lambda j: (0, 0)),
            pl.BlockSpec((Q_SIZE, 1), lambda j: (0, 0)),
        ],
        out_shape=[
            jax.ShapeDtypeStruct((Q_SIZE, 1), jnp.float32),
            jax.ShapeDtypeStruct((Q_SIZE, 1), jnp.int32),
        ],
        scratch_shapes=[
            pltpu.VMEM((Q_SIZE, 1), jnp.float32),
            pltpu.VMEM((Q_SIZE, 1), jnp.float32),
        ],
    )(xm2, xsq, idx_sq, index)
    return dist, arg


# speedup vs baseline: 4.8848x; 1.1131x over previous
import jax
import jax.numpy as jnp
from jax.experimental import pallas as pl
import jax.experimental.pallas.tpu as pltpu

Q_SIZE = 1024
K_SIZE = 100000
D_SIZE = 64
K_TILE = 4000
N_TILES = K_SIZE // K_TILE
CHUNK = 1024


def _nn_kernel(xm2_ref, xsq_ref, sq_ref, idx_ref, dist_ref, arg_ref,
               min_ref, argm_ref):
    j = pl.program_id(0)

    @pl.when(j == 0)
    def _init():
        min_ref[...] = jnp.full((Q_SIZE, 1), jnp.inf, jnp.float32)
        argm_ref[...] = jnp.zeros((Q_SIZE, 1), jnp.float32)

    tile = idx_ref[...]
    s = sq_ref[0]
    xsq = xsq_ref[...]
    xm2 = xm2_ref[...]

    run_min = min_ref[...]
    run_arg = argm_ref[...]
    for lo in range(0, K_TILE, CHUNK):
        width = min(CHUNK, K_TILE - lo)
        mm2 = jax.lax.dot_general(
            xm2, tile[lo:lo + width, :],
            dimension_numbers=(((1,), (1,)), ((), ())),
            preferred_element_type=jnp.float32,
        )
        t = xsq + s[:, lo:lo + width]
        d = t + mm2

        local_min = jnp.min(d, axis=1, keepdims=True)
        lane = jax.lax.broadcasted_iota(
            jnp.int32, (Q_SIZE, width), 1).astype(jnp.float32)
        local_arg = jnp.min(
            jnp.where(d == local_min, lane, jnp.float32(width)),
            axis=1, keepdims=True,
        ) + (j * K_TILE + lo).astype(jnp.float32)

        better = local_min < run_min
        run_min = jnp.where(better, local_min, run_min)
        run_arg = jnp.where(better, local_arg, run_arg)
    min_ref[...] = run_min
    argm_ref[...] = run_arg

    @pl.when(j == N_TILES - 1)
    def _done():
        dist_ref[...] = jnp.maximum(min_ref[...], 0.0)
        arg_ref[...] = argm_ref[...].astype(jnp.int32)


@jax.jit
def kernel(x, index):
    xsq = jnp.sum(x * x, axis=1, keepdims=True)
    xm2 = x * -2.0
    idx_sq = jnp.sum(index * index, axis=1).reshape(N_TILES, 1, K_TILE)
    dist, arg = pl.pallas_call(
        _nn_kernel,
        grid=(N_TILES,),
        in_specs=[
            pl.BlockSpec((Q_SIZE, D_SIZE), lambda j: (0, 0)),
            pl.BlockSpec((Q_SIZE, 1), lambda j: (0, 0)),
            pl.BlockSpec((1, 1, K_TILE), lambda j: (j, 0, 0)),
            pl.BlockSpec((K_TILE, D_SIZE), lambda j: (j, 0)),
        ],
        out_specs=[
            pl.BlockSpec((Q_SIZE, 1), lambda j: (0, 0)),
            pl.BlockSpec((Q_SIZE, 1), lambda j: (0, 0)),
        ],
        out_shape=[
            jax.ShapeDtypeStruct((Q_SIZE, 1), jnp.float32),
            jax.ShapeDtypeStruct((Q_SIZE, 1), jnp.int32),
        ],
        scratch_shapes=[
            pltpu.VMEM((Q_SIZE, 1), jnp.float32),
            pltpu.VMEM((Q_SIZE, 1), jnp.float32),
        ],
    )(xm2, xsq, idx_sq, index)
    return dist, arg


# speedup vs baseline: 4.9893x; 1.0214x over previous
import jax
import jax.numpy as jnp
from jax.experimental import pallas as pl
import jax.experimental.pallas.tpu as pltpu

Q_SIZE = 1024
K_SIZE = 100000
D_SIZE = 64
K_TILE = 10000
N_TILES = K_SIZE // K_TILE
CHUNK = 1024


def _nn_kernel(xm2_ref, xsq_ref, sq_ref, idx_ref, dist_ref, arg_ref,
               min_ref, argm_ref):
    j = pl.program_id(0)

    @pl.when(j == 0)
    def _init():
        min_ref[...] = jnp.full((Q_SIZE, 1), jnp.inf, jnp.float32)
        argm_ref[...] = jnp.zeros((Q_SIZE, 1), jnp.float32)

    tile = idx_ref[...]
    s = sq_ref[0]
    xsq = xsq_ref[...]
    xm2 = xm2_ref[...]

    run_min = min_ref[...]
    run_arg = argm_ref[...]
    for lo in range(0, K_TILE, CHUNK):
        width = min(CHUNK, K_TILE - lo)
        mm2 = jax.lax.dot_general(
            xm2, tile[lo:lo + width, :],
            dimension_numbers=(((1,), (1,)), ((), ())),
            preferred_element_type=jnp.float32,
        )
        t = xsq + s[:, lo:lo + width]
        d = t + mm2

        local_min = jnp.min(d, axis=1, keepdims=True)
        lane = jax.lax.broadcasted_iota(
            jnp.int32, (Q_SIZE, width), 1).astype(jnp.float32)
        local_arg = jnp.min(
            jnp.where(d == local_min, lane, jnp.float32(width)),
            axis=1, keepdims=True,
        ) + (j * K_TILE + lo).astype(jnp.float32)

        better = local_min < run_min
        run_min = jnp.where(better, local_min, run_min)
        run_arg = jnp.where(better, local_arg, run_arg)
    min_ref[...] = run_min
    argm_ref[...] = run_arg

    @pl.when(j == N_TILES - 1)
    def _done():
        dist_ref[...] = jnp.maximum(min_ref[...], 0.0)
        arg_ref[...] = argm_ref[...].astype(jnp.int32)


@jax.jit
def kernel(x, index):
    xsq = jnp.sum(x * x, axis=1, keepdims=True)
    xm2 = x * -2.0
    idx_sq = jnp.sum(index * index, axis=1).reshape(N_TILES, 1, K_TILE)
    dist, arg = pl.pallas_call(
        _nn_kernel,
        grid=(N_TILES,),
        in_specs=[
            pl.BlockSpec((Q_SIZE, D_SIZE), lambda j: (0, 0)),
            pl.BlockSpec((Q_SIZE, 1), lambda j: (0, 0)),
            pl.BlockSpec((1, 1, K_TILE), lambda j: (j, 0, 0)),
            pl.BlockSpec((K_TILE, D_SIZE), lambda j: (j, 0)),
        ],
        out_specs=[
            pl.BlockSpec((Q_SIZE, 1), lambda j: (0, 0)),
            pl.BlockSpec((Q_SIZE, 1), lambda j: (0, 0)),
        ],
        out_shape=[
            jax.ShapeDtypeStruct((Q_SIZE, 1), jnp.float32),
            jax.ShapeDtypeStruct((Q_SIZE, 1), jnp.int32),
        ],
        scratch_shapes=[
            pltpu.VMEM((Q_SIZE, 1), jnp.float32),
            pltpu.VMEM((Q_SIZE, 1), jnp.float32),
        ],
    )(xm2, xsq, idx_sq, index)
    return dist, arg


# speedup vs baseline: 5.5059x; 1.1036x over previous
import jax
import jax.numpy as jnp
from jax.experimental import pallas as pl
import jax.experimental.pallas.tpu as pltpu

Q_SIZE = 1024
K_SIZE = 100000
D_SIZE = 64
K_TILE = 10000
N_TILES = K_SIZE // K_TILE
CHUNK = 1024
LANES = 128
BIG = 3.0e7


def _nn_kernel(xm2_ref, xsq_ref, sq_ref, idx_ref, dist_ref, arg_ref,
               accv_ref, accb_ref):
    j = pl.program_id(0)

    @pl.when(j == 0)
    def _init():
        accv_ref[...] = jnp.full((Q_SIZE, LANES), jnp.inf, jnp.float32)
        accb_ref[...] = jnp.zeros((Q_SIZE, LANES), jnp.float32)

    tile = idx_ref[...]
    s = sq_ref[0]
    xsq = xsq_ref[...]
    xm2 = xm2_ref[...]

    accv = accv_ref[...]
    accb = accb_ref[...]
    for lo in range(0, K_TILE, CHUNK):
        width = min(CHUNK, K_TILE - lo)
        mm2 = jax.lax.dot_general(
            xm2, tile[lo:lo + width, :],
            dimension_numbers=(((1,), (1,)), ((), ())),
            preferred_element_type=jnp.float32,
        )
        t = xsq + s[:, lo:lo + width]
        d = t + mm2

        for b in range(width // LANES):
            dc = d[:, b * LANES:(b + 1) * LANES]
            start = (j * K_TILE + (lo + b * LANES)).astype(jnp.float32)
            m = dc < accv
            accv = jnp.where(m, dc, accv)
            accb = jnp.where(m, start, accb)
        w = width % LANES
        if w:
            base = width - w
            dtail = d[:, base:width]
            start = (j * K_TILE + (lo + base)).astype(jnp.float32)
            m = dtail < accv[:, :w]
            accv = jnp.concatenate(
                [jnp.where(m, dtail, accv[:, :w]), accv[:, w:]], axis=1)
            accb = jnp.concatenate(
                [jnp.where(m, start, accb[:, :w]), accb[:, w:]], axis=1)
    accv_ref[...] = accv
    accb_ref[...] = accb

    @pl.when(j == N_TILES - 1)
    def _done():
        vmin = jnp.min(accv, axis=1, keepdims=True)
        lane = jax.lax.broadcasted_iota(
            jnp.int32, (Q_SIZE, LANES), 1).astype(jnp.float32)
        gidx = accb + lane
        amin = jnp.min(
            jnp.where(accv == vmin, gidx, jnp.float32(BIG)), axis=1, keepdims=True)
        dist_ref[...] = jnp.maximum(vmin, 0.0)
        arg_ref[...] = amin.astype(jnp.int32)


@jax.jit
def kernel(x, index):
    xsq = jnp.sum(x * x, axis=1, keepdims=True)
    xm2 = x * -2.0
    idx_sq = jnp.sum(index * index, axis=1).reshape(N_TILES, 1, K_TILE)
    dist, arg = pl.pallas_call(
        _nn_kernel,
        grid=(N_TILES,),
        in_specs=[
            pl.BlockSpec((Q_SIZE, D_SIZE), lambda j: (0, 0)),
            pl.BlockSpec((Q_SIZE, 1), lambda j: (0, 0)),
            pl.BlockSpec((1, 1, K_TILE), lambda j: (j, 0, 0)),
            pl.BlockSpec((K_TILE, D_SIZE), lambda j: (j, 0)),
        ],
        out_specs=[
            pl.BlockSpec((Q_SIZE, 1), lambda j: (0, 0)),
            pl.BlockSpec((Q_SIZE, 1), lambda j: (0, 0)),
        ],
        out_shape=[
            jax.ShapeDtypeStruct((Q_SIZE, 1), jnp.float32),
            jax.ShapeDtypeStruct((Q_SIZE, 1), jnp.int32),
        ],
        scratch_shapes=[
            pltpu.VMEM((Q_SIZE, LANES), jnp.float32),
            pltpu.VMEM((Q_SIZE, LANES), jnp.float32),
        ],
    )(xm2, xsq, idx_sq, index)
    return dist, arg
